# baseline (speedup 1.0000x reference)
import jax
import jax.numpy as jnp
from jax.experimental import pallas as pl

B, N, F, ED, OUT0, OUT1, DMAX = 512, 30, 128, 16, 128, 128, 13
NP = 32
BT = 16
GRID = B // BT
ROWS = BT * NP
_PREC = jax.lax.Precision.DEFAULT


def _bmm(g, h):
    return jax.lax.dot_general(
        g, h, (((2,), (1,)), ((0,), (0,))), precision=_PREC)


def _degree_update(m, Hq, masks, valid):
    zsel = jnp.zeros((m.shape[0], OUT0), jnp.float32)
    for k in range(7):
        z = jnp.dot(m, Hq[k], precision=_PREC)
        zsel = zsel + masks[2 * k] * z[:, :OUT0]
        if 2 * k + 1 < DMAX:
            zsel = zsel + masks[2 * k + 1] * z[:, OUT0:]
    return jax.nn.sigmoid(zsel) * valid


def _readout(hf, W):
    a = jnp.dot(hf, W, precision=_PREC)
    amax = jnp.max(a, axis=-1, keepdims=True)
    ex = jnp.exp(a - amax)
    sm = ex / jnp.sum(ex, axis=-1, keepdims=True)
    nz = (amax > 0) | (jnp.min(a, axis=-1, keepdims=True) < 0)
    sm = jnp.where(nz, sm, 0.0)
    return jnp.sum(sm.reshape(BT, NP, OUT1), axis=1)


def _fused_kernel(g_ref, h_ref, e4_ref, H0q_ref, H1q_ref,
                  W0_ref, W1_ref, W2_ref, nw0_ref, nb0_ref, nw1_ref,
                  nb1_ref, nw2_ref, nb2_ref, nw3_ref, nb3_ref, out_ref):
    g3 = jnp.pad(g_ref[...], ((0, 0), (0, NP - N), (0, NP - N)))
    hp = jnp.pad(h_ref[...], ((0, 0), (0, NP - N), (0, 0)))

    deg = jnp.sum(g3.reshape(ROWS, NP), axis=1, keepdims=True)
    row_v = jax.lax.broadcasted_iota(jnp.int32, (ROWS, 1), 0) % NP
    valid = ((deg < DMAX) & (row_v < N)).astype(jnp.float32)
    masks = [(deg == i).astype(jnp.float32) for i in range(DMAX)]

    e4 = e4_ref[...].reshape(BT, N, N, ED)
    m_e = jnp.sum(g_ref[...][..., None] * e4, axis=2)
    m_e32 = jnp.pad(m_e, ((0, 0), (0, NP - N), (0, 0))).reshape(ROWS, ED)

    mh1 = _bmm(g3, hp).reshape(ROWS, F)
    m1 = jnp.concatenate([mh1, m_e32], axis=1)
    h1 = _degree_update(m1, H0q_ref[...], masks, valid)
    mh2 = _bmm(g3, h1.reshape(BT, NP, OUT0)).reshape(ROWS, OUT0)
    m2 = jnp.concatenate([mh2, m_e32], axis=1)
    h2 = _degree_update(m2, H1q_ref[...], masks, valid)

    acc = (_readout(hp.reshape(ROWS, F), W0_ref[...])
           + _readout(h1, W1_ref[...])
           + _readout(h2, W2_ref[...]))

    x = jax.nn.relu(jnp.dot(acc, nw0_ref[...], precision=_PREC)
                    + nb0_ref[...])
    x = jax.nn.relu(jnp.dot(x, nw1_ref[...], precision=_PREC)
                    + nb1_ref[...])
    x = jax.nn.relu(jnp.dot(x, nw2_ref[...], precision=_PREC)
                    + nb2_ref[...])
    out_ref[...] = (jnp.dot(x, nw3_ref[...], precision=_PREC)
                    + nb3_ref[...])


@jax.jit
def kernel(g, h_in, e, H0, H1, W0, W1, W2, nw0, nb0, nw1, nb1, nw2, nb2,
           nw3, nb3):
    nb0r, nb1r, nb2r, nb3r = (x.reshape(1, -1) for x in (nb0, nb1, nb2, nb3))
    H0e = jnp.concatenate([H0, jnp.zeros((1, F + ED, OUT0), jnp.float32)], 0)
    H1e = jnp.concatenate([H1, jnp.zeros((1, OUT0 + ED, OUT1), jnp.float32)], 0)
    H0q = jnp.concatenate([H0e[0::2], H0e[1::2]], axis=2)
    H1q = jnp.concatenate([H1e[0::2], H1e[1::2]], axis=2)

    tile3 = lambda i: (i, 0, 0)
    tile2 = lambda i: (i, 0)
    rep2 = lambda i: (0, 0)

    out = pl.pallas_call(
        _fused_kernel,
        grid=(GRID,),
        in_specs=[
            pl.BlockSpec((BT, N, N), tile3),
            pl.BlockSpec((BT, N, F), tile3),
            pl.BlockSpec((BT * N, N, ED), lambda i: (i, 0, 0)),
            pl.BlockSpec((7, F + ED, 2 * OUT0), lambda i: (0, 0, 0)),
            pl.BlockSpec((7, OUT0 + ED, 2 * OUT1), lambda i: (0, 0, 0)),
            pl.BlockSpec((F, OUT1), rep2),
            pl.BlockSpec((OUT0, OUT1), rep2),
            pl.BlockSpec((OUT1, OUT1), rep2),
            pl.BlockSpec((OUT1, 128), rep2),
            pl.BlockSpec((1, 128), rep2),
            pl.BlockSpec((128, 256), rep2),
            pl.BlockSpec((1, 256), rep2),
            pl.BlockSpec((256, 128), rep2),
            pl.BlockSpec((1, 128), rep2),
            pl.BlockSpec((128, 1), rep2),
            pl.BlockSpec((1, 1), rep2),
        ],
        out_specs=pl.BlockSpec((BT, 1), tile2),
        out_shape=jax.ShapeDtypeStruct((B, 1), jnp.float32),
    )(g, h_in, e.reshape(B * N, N, ED), H0q, H1q, W0, W1, W2, nw0, nb0r, nw1, nb1r, nw2,
      nb2r, nw3, nb3r)
    return out


# speedup vs baseline: 1.1025x; 1.1025x over previous
import jax
import jax.numpy as jnp
from jax.experimental import pallas as pl

B, N, F, ED, OUT0, OUT1, DMAX = 512, 30, 128, 16, 128, 128, 13
NP = 32
BT = 32
GRID = B // BT
ROWS = BT * NP
_PREC = jax.lax.Precision.DEFAULT


def _bmm(g, h):
    return jax.lax.dot_general(
        g, h, (((2,), (1,)), ((0,), (0,))), precision=_PREC)


def _degree_update(m, Hq, masks, valid):
    zsel = jnp.zeros((m.shape[0], OUT0), jnp.float32)
    for k in range(7):
        z = jnp.dot(m, Hq[k], precision=_PREC)
        zsel = zsel + masks[2 * k] * z[:, :OUT0]
        if 2 * k + 1 < DMAX:
            zsel = zsel + masks[2 * k + 1] * z[:, OUT0:]
    return jax.nn.sigmoid(zsel) * valid


def _readout(hf, W):
    a = jnp.dot(hf, W, precision=_PREC)
    amax = jnp.max(a, axis=-1, keepdims=True)
    ex = jnp.exp(a - amax)
    sm = ex / jnp.sum(ex, axis=-1, keepdims=True)
    nz = (amax > 0) | (jnp.min(a, axis=-1, keepdims=True) < 0)
    sm = jnp.where(nz, sm, 0.0)
    return jnp.sum(sm.reshape(BT, NP, OUT1), axis=1)


def _fused_kernel(g_ref, h_ref, ea_ref, eb_ref, H0q_ref, H1q_ref,
                  W0_ref, W1_ref, W2_ref, nw0_ref, nb0_ref, nw1_ref,
                  nb1_ref, nw2_ref, nb2_ref, nw3_ref, nb3_ref, out_ref):
    g3 = jnp.pad(g_ref[...], ((0, 0), (0, NP - N), (0, NP - N)))
    hp = jnp.pad(h_ref[...], ((0, 0), (0, NP - N), (0, 0)))

    deg = jnp.sum(g3.reshape(ROWS, NP), axis=1, keepdims=True)
    row_v = jax.lax.broadcasted_iota(jnp.int32, (ROWS, 1), 0) % NP
    valid = ((deg < DMAX) & (row_v < N)).astype(jnp.float32)
    masks = [(deg == i).astype(jnp.float32) for i in range(DMAX)]

    ea = ea_ref[...].reshape(BT // 2, N, N, ED)
    eb = eb_ref[...].reshape(BT // 2, N, N, ED)
    gx = g_ref[...][..., None]
    m_e = jnp.concatenate(
        [jnp.sum(gx[:BT // 2] * ea, axis=2),
         jnp.sum(gx[BT // 2:] * eb, axis=2)], axis=0)
    m_e32 = jnp.pad(m_e, ((0, 0), (0, NP - N), (0, 0))).reshape(ROWS, ED)

    mh1 = _bmm(g3, hp).reshape(ROWS, F)
    m1 = jnp.concatenate([mh1, m_e32], axis=1)
    h1 = _degree_update(m1, H0q_ref[...], masks, valid)
    mh2 = _bmm(g3, h1.reshape(BT, NP, OUT0)).reshape(ROWS, OUT0)
    m2 = jnp.concatenate([mh2, m_e32], axis=1)
    h2 = _degree_update(m2, H1q_ref[...], masks, valid)

    acc = (_readout(hp.reshape(ROWS, F), W0_ref[...])
           + _readout(h1, W1_ref[...])
           + _readout(h2, W2_ref[...]))

    x = jax.nn.relu(jnp.dot(acc, nw0_ref[...], precision=_PREC)
                    + nb0_ref[...])
    x = jax.nn.relu(jnp.dot(x, nw1_ref[...], precision=_PREC)
                    + nb1_ref[...])
    x = jax.nn.relu(jnp.dot(x, nw2_ref[...], precision=_PREC)
                    + nb2_ref[...])
    out_ref[...] = (jnp.dot(x, nw3_ref[...], precision=_PREC)
                    + nb3_ref[...])


@jax.jit
def kernel(g, h_in, e, H0, H1, W0, W1, W2, nw0, nb0, nw1, nb1, nw2, nb2,
           nw3, nb3):
    nb0r, nb1r, nb2r, nb3r = (x.reshape(1, -1) for x in (nb0, nb1, nb2, nb3))
    H0e = jnp.concatenate([H0, jnp.zeros((1, F + ED, OUT0), jnp.float32)], 0)
    H1e = jnp.concatenate([H1, jnp.zeros((1, OUT0 + ED, OUT1), jnp.float32)], 0)
    H0q = jnp.concatenate([H0e[0::2], H0e[1::2]], axis=2)
    H1q = jnp.concatenate([H1e[0::2], H1e[1::2]], axis=2)

    e3 = e.reshape(B * N, N, ED)
    tile3 = lambda i: (i, 0, 0)
    tile2 = lambda i: (i, 0)
    rep2 = lambda i: (0, 0)

    out = pl.pallas_call(
        _fused_kernel,
        grid=(GRID,),
        in_specs=[
            pl.BlockSpec((BT, N, N), tile3),
            pl.BlockSpec((BT, N, F), tile3),
            pl.BlockSpec((BT * N // 2, N, ED), lambda i: (2 * i, 0, 0)),
            pl.BlockSpec((BT * N // 2, N, ED), lambda i: (2 * i + 1, 0, 0)),
            pl.BlockSpec((7, F + ED, 2 * OUT0), lambda i: (0, 0, 0)),
            pl.BlockSpec((7, OUT0 + ED, 2 * OUT1), lambda i: (0, 0, 0)),
            pl.BlockSpec((F, OUT1), rep2),
            pl.BlockSpec((OUT0, OUT1), rep2),
            pl.BlockSpec((OUT1, OUT1), rep2),
            pl.BlockSpec((OUT1, 128), rep2),
            pl.BlockSpec((1, 128), rep2),
            pl.BlockSpec((128, 256), rep2),
            pl.BlockSpec((1, 256), rep2),
            pl.BlockSpec((256, 128), rep2),
            pl.BlockSpec((1, 128), rep2),
            pl.BlockSpec((128, 1), rep2),
            pl.BlockSpec((1, 1), rep2),
        ],
        out_specs=pl.BlockSpec((BT, 1), tile2),
        out_shape=jax.ShapeDtypeStruct((B, 1), jnp.float32),
    )(g, h_in, e3, e3, H0q, H1q, W0, W1, W2, nw0, nb0r, nw1, nb1r, nw2,
      nb2r, nw3, nb3r)
    return out


# speedup vs baseline: 1.1052x; 1.0024x over previous
import jax
import jax.numpy as jnp
from jax.experimental import pallas as pl

B, N, F, ED, OUT0, OUT1, DMAX = 512, 30, 128, 16, 128, 128, 13
NP = 32
BT = 32
GRID = B // BT
ROWS = BT * NP
_PREC = jax.lax.Precision.DEFAULT


def _bmm(g, h):
    return jax.lax.dot_general(
        g, h, (((2,), (1,)), ((0,), (0,))), precision=_PREC)


def _degree_update(m, Hq, masks, valid):
    zsel = jnp.zeros((m.shape[0], OUT0), jnp.float32)
    for k in range(7):
        z = jnp.dot(m, Hq[k], precision=_PREC)
        zsel = zsel + masks[2 * k] * z[:, :OUT0]
        if 2 * k + 1 < DMAX:
            zsel = zsel + masks[2 * k + 1] * z[:, OUT0:]
    return jax.nn.sigmoid(zsel) * valid


def _readout(hf, W):
    a = jnp.dot(hf, W, precision=_PREC)
    amax = jnp.max(a, axis=-1, keepdims=True)
    ex = jnp.exp(a - amax)
    sm = ex / jnp.sum(ex, axis=-1, keepdims=True)
    nz = (amax > 0) | (jnp.min(a, axis=-1, keepdims=True) < 0)
    sm = jnp.where(nz, sm, 0.0)
    return jnp.sum(sm.reshape(BT, NP, OUT1), axis=1)


def _fused_kernel(g_ref, h_ref, e3_ref, H0q_ref, H1q_ref,
                  W0_ref, W1_ref, W2_ref, nw0_ref, nb0_ref, nw1_ref,
                  nb1_ref, nw2_ref, nb2_ref, nw3_ref, nb3_ref, out_ref):
    g3 = jnp.pad(g_ref[...], ((0, 0), (0, NP - N), (0, NP - N)))
    hp = jnp.pad(h_ref[...], ((0, 0), (0, NP - N), (0, 0)))

    deg = jnp.sum(g3.reshape(ROWS, NP), axis=1, keepdims=True)
    row_v = jax.lax.broadcasted_iota(jnp.int32, (ROWS, 1), 0) % NP
    valid = ((deg < DMAX) & (row_v < N)).astype(jnp.float32)
    masks = [(deg == i).astype(jnp.float32) for i in range(DMAX)]

    e4 = e3_ref[...].reshape(BT, N, N, ED)
    m_e = jnp.sum(g_ref[...][..., None] * e4, axis=2)
    m_e32 = jnp.pad(m_e, ((0, 0), (0, NP - N), (0, 0))).reshape(ROWS, ED)

    mh1 = _bmm(g3, hp).reshape(ROWS, F)
    m1 = jnp.concatenate([mh1, m_e32], axis=1)
    h1 = _degree_update(m1, H0q_ref[...], masks, valid)
    mh2 = _bmm(g3, h1.reshape(BT, NP, OUT0)).reshape(ROWS, OUT0)
    m2 = jnp.concatenate([mh2, m_e32], axis=1)
    h2 = _degree_update(m2, H1q_ref[...], masks, valid)

    acc = (_readout(hp.reshape(ROWS, F), W0_ref[...])
           + _readout(h1, W1_ref[...])
           + _readout(h2, W2_ref[...]))

    x = jax.nn.relu(jnp.dot(acc, nw0_ref[...], precision=_PREC)
                    + nb0_ref[...])
    x = jax.nn.relu(jnp.dot(x, nw1_ref[...], precision=_PREC)
                    + nb1_ref[...])
    x = jax.nn.relu(jnp.dot(x, nw2_ref[...], precision=_PREC)
                    + nb2_ref[...])
    out_ref[...] = (jnp.dot(x, nw3_ref[...], precision=_PREC)
                    + nb3_ref[...])


@jax.jit
def kernel(g, h_in, e, H0, H1, W0, W1, W2, nw0, nb0, nw1, nb1, nw2, nb2,
           nw3, nb3):
    nb0r, nb1r, nb2r, nb3r = (x.reshape(1, -1) for x in (nb0, nb1, nb2, nb3))
    H0e = jnp.concatenate([H0, jnp.zeros((1, F + ED, OUT0), jnp.float32)], 0)
    H1e = jnp.concatenate([H1, jnp.zeros((1, OUT0 + ED, OUT1), jnp.float32)], 0)
    H0q = jnp.concatenate([H0e[0::2], H0e[1::2]], axis=2)
    H1q = jnp.concatenate([H1e[0::2], H1e[1::2]], axis=2)

    e3 = e.reshape(B * N, N, ED)
    tile3 = lambda i: (i, 0, 0)
    tile2 = lambda i: (i, 0)
    rep2 = lambda i: (0, 0)

    out = pl.pallas_call(
        _fused_kernel,
        grid=(GRID,),
        in_specs=[
            pl.BlockSpec((BT, N, N), tile3),
            pl.BlockSpec((BT, N, F), tile3),
            pl.BlockSpec((BT * N, N, ED), lambda i: (i, 0, 0)),
            pl.BlockSpec((7, F + ED, 2 * OUT0), lambda i: (0, 0, 0)),
            pl.BlockSpec((7, OUT0 + ED, 2 * OUT1), lambda i: (0, 0, 0)),
            pl.BlockSpec((F, OUT1), rep2),
            pl.BlockSpec((OUT0, OUT1), rep2),
            pl.BlockSpec((OUT1, OUT1), rep2),
            pl.BlockSpec((OUT1, 128), rep2),
            pl.BlockSpec((1, 128), rep2),
            pl.BlockSpec((128, 256), rep2),
            pl.BlockSpec((1, 256), rep2),
            pl.BlockSpec((256, 128), rep2),
            pl.BlockSpec((1, 128), rep2),
            pl.BlockSpec((128, 1), rep2),
            pl.BlockSpec((1, 1), rep2),
        ],
        out_specs=pl.BlockSpec((BT, 1), tile2),
        out_shape=jax.ShapeDtypeStruct((B, 1), jnp.float32),
    )(g, h_in, e3, H0q, H1q, W0, W1, W2, nw0, nb0r, nw1, nb1r, nw2,
      nb2r, nw3, nb3r)
    return out
